# baseline (speedup 1.0000x reference)
import functools
import math

import jax
import jax.numpy as jnp
from jax import lax
from jax.experimental import pallas as pl
from jax.experimental.pallas import tpu as pltpu
from jax.experimental.pallas import tpu_sc as plsc

LANES = 16
C = 128
NC = 2
NS = 16
NW = NC * NS


def _proj_body(x_ref, wq_ref, bq_ref, wk_ref, bk_ref, wv_ref, bv_ref,
               qb_ref, kb_ref, v_ref):
    dn = (((1,), (1,)), ((), ()))
    xb = x_ref[...]
    q = lax.dot_general(xb, wq_ref[...], dn,
                        preferred_element_type=jnp.float32,
                        precision=lax.Precision.HIGHEST) + bq_ref[...]
    k = lax.dot_general(xb, wk_ref[...], dn,
                        preferred_element_type=jnp.float32,
                        precision=lax.Precision.HIGHEST) + bk_ref[...]
    qb_ref[...] = q.astype(jnp.bfloat16)
    kb_ref[...] = k.astype(jnp.bfloat16)
    v_ref[...] = lax.dot_general(xb, wv_ref[...], dn,
                                 preferred_element_type=jnp.float32,
                                 precision=lax.Precision.HIGHEST) + bv_ref[...]


def _project(x, W_q, b_q, W_k, b_k, W_v, b_v):
    n, f_in = x.shape
    h = W_q.shape[0]
    blk = 2000
    grid = (n // blk,)
    w_spec = pl.BlockSpec((h, f_in), lambda i: (0, 0))
    b_spec = pl.BlockSpec((1, h), lambda i: (0, 0))
    row_spec = pl.BlockSpec((blk, f_in), lambda i: (i, 0))
    out_spec = pl.BlockSpec((blk, h), lambda i: (i, 0))
    return pl.pallas_call(
        _proj_body,
        grid=grid,
        in_specs=[row_spec, w_spec, b_spec, w_spec, b_spec, w_spec, b_spec],
        out_specs=[out_spec, out_spec, out_spec],
        out_shape=[jax.ShapeDtypeStruct((n, h), jnp.bfloat16),
                   jax.ShapeDtypeStruct((n, h), jnp.bfloat16),
                   jax.ShapeDtypeStruct((n, h), jnp.float32)],
    )(x, W_q, b_q.reshape(1, h), W_k, b_k.reshape(1, h), W_v, b_v.reshape(1, h))


def _make_phase1(n, e, h, g):
    nchunk = e // C
    npair = nchunk // 2
    ppt = -(-npair // NW)
    if ppt % 2:
        ppt += 1
    nsuper = ppt // 2
    base_n = npair // NW
    rem = npair % NW
    mesh = plsc.VectorSubcoreMesh(core_axis_name="c", subcore_axis_name="s")
    inv_sqrt_h = 1.0 / math.sqrt(h)

    @functools.partial(
        pl.kernel,
        out_type=[jax.ShapeDtypeStruct((e,), jnp.float32),
                  jax.ShapeDtypeStruct((NW, g), jnp.float32)],
        mesh=mesh,
        scratch_types=[
            pltpu.VMEM((2 * C,), jnp.int32),
            pltpu.VMEM((2 * C,), jnp.int32),
            pltpu.VMEM((2 * C,), jnp.int32),
            pltpu.VMEM((2 * C,), jnp.int32),
            pltpu.VMEM((C, h // 2), jnp.int32),
            pltpu.VMEM((C, h // 2), jnp.int32),
            pltpu.VMEM((C, h // 2), jnp.int32),
            pltpu.VMEM((C, h // 2), jnp.int32),
            pltpu.VMEM((2 * C,), jnp.float32),
            pltpu.VMEM((2 * C,), jnp.float32),
            pltpu.VMEM((n,), jnp.int32),
            pltpu.VMEM((g * LANES,), jnp.float32),
            pltpu.VMEM((g,), jnp.float32),
            pltpu.VMEM((LANES * LANES,), jnp.float32),
            pltpu.SemaphoreType.DMA,
            pltpu.SemaphoreType.DMA,
            pltpu.SemaphoreType.DMA,
            pltpu.SemaphoreType.DMA,
        ],
        compiler_params=pltpu.CompilerParams(needs_layout_passes=False,
                                             use_tc_tiling_on_sc=False),
    )
    def phase1(q_hbm, k_hbm, src_hbm, dst_hbm, batch_hbm,
               expv_hbm, denom_hbm,
               si0, si1, di0, di1, qv0, qv1, kv0, kv1, ev0, ev1,
               batch_v, dacc_v, dred_v, dots_v,
               sq0, sq1, sk0, sk1):
        si = [si0, si1]
        di = [di0, di1]
        qv = [qv0, qv1]
        kv = [kv0, kv1]
        ev = [ev0, ev1]
        sq = [sq0, sq1]
        sk = [sk0, sk1]
        cid = lax.axis_index("c")
        sid = lax.axis_index("s")
        wid = sid * NC + cid
        iota = jnp.arange(LANES, dtype=jnp.int32)
        zeros16 = jnp.zeros((LANES,), jnp.float32)

        pltpu.sync_copy(batch_hbm, batch_v)

        def zero_body(i, _):
            dacc_v[pl.ds(i * LANES, LANES)] = zeros16
            return _
        lax.fori_loop(0, g, zero_body, None)

        base_pair = wid * base_n + jnp.minimum(wid, rem)
        n_real = jnp.where(wid < rem, base_n + 1, base_n)

        def load_pair_idx(pj, b):
            pc = jnp.minimum(pj, npair - 1)
            pltpu.sync_copy(src_hbm.at[pl.ds(pc * 2 * C, 2 * C)], si[b])
            pltpu.sync_copy(dst_hbm.at[pl.ds(pc * 2 * C, 2 * C)], di[b])

        def issue_gather(b, hh):
            pltpu.async_copy(q_hbm.at[di[b].at[pl.ds(hh * C, C)]],
                             qv[hh], sq[hh])
            pltpu.async_copy(k_hbm.at[si[b].at[pl.ds(hh * C, C)]],
                             kv[hh], sk[hh])

        def wait_gather(b, hh):
            pltpu.make_async_copy(q_hbm.at[di[b].at[pl.ds(hh * C, C)]],
                                  qv[hh], sq[hh]).wait()
            pltpu.make_async_copy(k_hbm.at[si[b].at[pl.ds(hh * C, C)]],
                                  kv[hh], sk[hh]).wait()

        load_pair_idx(base_pair, 0)
        issue_gather(0, 0)

        def super_body(j, _):
            for c in range(4):
                b, hh = c // 2, c % 2
                pair_loc = j * 2 + b
                pair_g = base_pair + pair_loc
                if c == 0:
                    issue_gather(0, 1)
                elif c == 1:
                    load_pair_idx(pair_g + 1, 1)
                    issue_gather(1, 0)
                elif c == 2:
                    issue_gather(1, 1)
                else:
                    @pl.when(j < nsuper - 1)
                    def _prefetch():
                        load_pair_idx(pair_g + 1, 0)
                        issue_gather(0, 0)
                wait_gather(b, hh)
                validf = jnp.where(pair_loc < n_real, 1.0, 0.0)
                vbcast = jnp.full((LANES,), 1.0, jnp.float32) * validf

                def blk_body(blk, _b):
                    for l in range(LANES):
                        ei = blk * LANES + l
                        acc = zeros16
                        for hb in range(h // (2 * LANES)):
                            qi = qv[hh][ei, pl.ds(hb * LANES, LANES)]
                            ki = kv[hh][ei, pl.ds(hb * LANES, LANES)]
                            qlo = plsc.bitcast(qi << 16, jnp.float32)
                            klo = plsc.bitcast(ki << 16, jnp.float32)
                            qhi = plsc.bitcast(qi & jnp.int32(-65536),
                                               jnp.float32)
                            khi = plsc.bitcast(ki & jnp.int32(-65536),
                                               jnp.float32)
                            acc = acc + qlo * klo + qhi * khi
                        dots_v[pl.ds(l * LANES, LANES)] = acc
                    dot = zeros16
                    for jj in range(LANES):
                        dot = dot + plsc.load_gather(dots_v,
                                                     [iota * LANES + jj])
                    sc = jnp.exp(dot * inv_sqrt_h)
                    ev[b][pl.ds(hh * C + blk * LANES, LANES)] = sc
                    src16 = si[b][pl.ds(hh * C + blk * LANES, LANES)]
                    bse16 = plsc.load_gather(batch_v, [src16])
                    plsc.addupdate_scatter(dacc_v, [bse16 * LANES + iota],
                                           sc * vbcast)
                    return _b
                lax.fori_loop(0, C // LANES, blk_body, None)
                if hh == 1:
                    pc = jnp.minimum(pair_g, npair - 1)
                    pltpu.sync_copy(ev[b],
                                    expv_hbm.at[pl.ds(pc * 2 * C, 2 * C)])
            return _
        lax.fori_loop(0, nsuper, super_body, None)

        for blk in range(g // LANES):
            acc = zeros16
            for l in range(LANES):
                acc = acc + plsc.load_gather(
                    dacc_v, [(iota + blk * LANES) * LANES + l])
            dred_v[pl.ds(blk * LANES, LANES)] = acc
        pltpu.sync_copy(dred_v, denom_hbm.at[wid])

    return phase1


def _make_phase2(n, e, h, g):
    nchunk = e // C
    ROWCH = 80
    mesh = plsc.VectorSubcoreMesh(core_axis_name="c", subcore_axis_name="s")

    npair = nchunk // 2
    ppt = -(-npair // NW)
    if ppt % 2:
        ppt += 1
    nsuper = ppt // 2
    base_n = npair // NW
    rem = npair % NW

    @functools.partial(
        pl.kernel,
        out_type=jax.ShapeDtypeStruct((NC, n, h), jnp.float32),
        mesh=mesh,
        scratch_types=[
            pltpu.VMEM((2 * C,), jnp.int32),
            pltpu.VMEM((2 * C,), jnp.int32),
            pltpu.VMEM((2 * C,), jnp.int32),
            pltpu.VMEM((2 * C,), jnp.int32),
            pltpu.VMEM((C,), jnp.int32),
            pltpu.VMEM((C,), jnp.int32),
            pltpu.VMEM((C, h), jnp.float32),
            pltpu.VMEM((C, h), jnp.float32),
            pltpu.VMEM((2 * C,), jnp.float32),
            pltpu.VMEM((2 * C,), jnp.float32),
            pltpu.VMEM((C,), jnp.float32),
            pltpu.VMEM((C,), jnp.float32),
            pltpu.VMEM((n,), jnp.int32),
            pltpu.VMEM((NW, g), jnp.float32),
            pltpu.VMEM((g,), jnp.float32),
            pltpu.VMEM_SHARED((n, h), jnp.float32),
            pltpu.SemaphoreType.DMA,
            pltpu.SemaphoreType.DMA,
            pltpu.SemaphoreType.DMA,
            pltpu.SemaphoreType.DMA,
        ],
        compiler_params=pltpu.CompilerParams(needs_layout_passes=False),
    )
    def phase2(v_hbm, src_hbm, dst_hbm, expv_hbm, batch_hbm, denom_hbm,
               zeros_hbm, part_hbm,
               si0, si1, di0, di1, db0, db1, vv0, vv1, ev0, ev1, av0, av1,
               batch_v, dpart_v, rec_v,
               acc_sh, sv0, sv1, ss0, ss1):
        si = [si0, si1]
        di = [di0, di1]
        db = [db0, db1]
        vv = [vv0, vv1]
        ev = [ev0, ev1]
        av = [av0, av1]
        sv = [sv0, sv1]
        ss = [ss0, ss1]
        cid = lax.axis_index("c")
        sid = lax.axis_index("s")
        wid = sid * NC + cid
        iota = jnp.arange(LANES, dtype=jnp.int32)
        zeros16 = jnp.zeros((LANES,), jnp.float32)

        pltpu.sync_copy(batch_hbm, batch_v)
        pltpu.sync_copy(denom_hbm, dpart_v)
        for blk in range(g // LANES):
            acc = zeros16
            for w in range(NW):
                acc = acc + dpart_v[w, pl.ds(blk * LANES, LANES)]
            rec_v[pl.ds(blk * LANES, LANES)] = 1.0 / (acc + 1e-6)

        nrch = n // ROWCH
        n_z = (nrch - sid + NS - 1) // NS

        def zero_body(i, _):
            r0 = (sid + i * NS) * ROWCH
            pltpu.sync_copy(zeros_hbm.at[pl.ds(r0, ROWCH)],
                            acc_sh.at[pl.ds(r0, ROWCH)])
            return _
        lax.fori_loop(0, n_z, zero_body, None)
        plsc.subcore_barrier()

        base_pair = wid * base_n + jnp.minimum(wid, rem)
        n_real = jnp.where(wid < rem, base_n + 1, base_n)

        def load_pair(pj, b):
            pc = jnp.minimum(pj, npair - 1)
            pltpu.sync_copy(src_hbm.at[pl.ds(pc * 2 * C, 2 * C)], si[b])
            pltpu.sync_copy(dst_hbm.at[pl.ds(pc * 2 * C, 2 * C)], di[b])
            pltpu.sync_copy(expv_hbm.at[pl.ds(pc * 2 * C, 2 * C)], ev[b])

        def issue_vgather(b, hh):
            pltpu.async_copy(v_hbm.at[si[b].at[pl.ds(hh * C, C)]],
                             vv[hh], sv[hh])

        def wait_vgather(b, hh):
            pltpu.make_async_copy(v_hbm.at[si[b].at[pl.ds(hh * C, C)]],
                                  vv[hh], sv[hh]).wait()

        def issue_scatter(hh):
            pltpu.async_copy(vv[hh], acc_sh.at[db[hh]], ss[hh], add=True)

        def wait_scatter(hh):
            pltpu.make_async_copy(vv[hh], acc_sh.at[db[hh]], ss[hh]).wait()

        load_pair(base_pair, 0)
        issue_vgather(0, 0)

        def super_body(j, _):
            for c in range(4):
                b, hh = c // 2, c % 2
                pair_loc = j * 2 + b
                pair_g = base_pair + pair_loc
                if c == 0:
                    @pl.when(j > 0)
                    def _drain():
                        wait_scatter(1)
                    issue_vgather(0, 1)
                elif c == 1:
                    wait_scatter(0)
                    load_pair(pair_g + 1, 1)
                    issue_vgather(1, 0)
                elif c == 2:
                    wait_scatter(1)
                    issue_vgather(1, 1)
                else:
                    @pl.when(j < nsuper - 1)
                    def _prefetch():
                        wait_scatter(0)
                        load_pair(pair_g + 1, 0)
                        issue_vgather(0, 0)
                wait_vgather(b, hh)
                validf = jnp.where(pair_loc < n_real, 1.0, 0.0)
                vbcast = jnp.full((LANES,), 1.0, jnp.float32) * validf
                for blk in range(C // LANES):
                    off = hh * C + blk * LANES
                    src16 = si[b][pl.ds(off, LANES)]
                    bse16 = plsc.load_gather(batch_v, [src16])
                    rd16 = plsc.load_gather(rec_v, [bse16])
                    av[hh][pl.ds(blk * LANES, LANES)] = \
                        ev[b][pl.ds(off, LANES)] * rd16 * vbcast
                    db[hh][pl.ds(blk * LANES, LANES)] = di[b][pl.ds(off, LANES)]

                def scale_body(blk, _s):
                    av16 = av[hh][pl.ds(blk * LANES, LANES)]
                    for l in range(LANES):
                        ei = blk * LANES + l
                        a_s = av16[l]
                        for hb in range(h // LANES):
                            vv[hh][ei, pl.ds(hb * LANES, LANES)] = \
                                vv[hh][ei, pl.ds(hb * LANES, LANES)] * a_s
                    return _s
                lax.fori_loop(0, C // LANES, scale_body, None)
                issue_scatter(hh)
            return _
        lax.fori_loop(0, nsuper, super_body, None)

        wait_scatter(0)
        wait_scatter(1)
        plsc.subcore_barrier()

        def out_body(i, _):
            r0 = (sid + i * NS) * ROWCH
            pltpu.sync_copy(acc_sh.at[pl.ds(r0, ROWCH)],
                            part_hbm.at[cid, pl.ds(r0, ROWCH)])
            return _
        lax.fori_loop(0, n_z, out_body, None)

    return phase2


def _sum_body(a_ref, b_ref, o_ref):
    o_ref[...] = a_ref[...] + b_ref[...]


def _sum_parts(p0, p1):
    n, h = p0.shape
    blk = 2000
    spec = pl.BlockSpec((blk, h), lambda i: (i, 0))
    return pl.pallas_call(
        _sum_body,
        grid=(n // blk,),
        in_specs=[spec, spec],
        out_specs=spec,
        out_shape=jax.ShapeDtypeStruct((n, h), jnp.float32),
    )(p0, p1)


def kernel(x, edge_index, batch, W_q, b_q, W_k, b_k, W_v, b_v):
    n, f_in = x.shape
    e = edge_index.shape[1]
    h = W_q.shape[0]
    g = 64

    q_bf, k_bf, v = _project(x, W_q, b_q, W_k, b_k, W_v, b_v)
    q32 = lax.bitcast_convert_type(q_bf.reshape(n, h // 2, 2), jnp.int32)
    k32 = lax.bitcast_convert_type(k_bf.reshape(n, h // 2, 2), jnp.int32)
    src = edge_index[0]
    dst = edge_index[1]

    expv, denom_part = _make_phase1(n, e, h, g)(q32, k32, src, dst, batch)
    zeros = jnp.zeros((n, h), jnp.float32)
    parts = _make_phase2(n, e, h, g)(v, src, dst, expv, batch, denom_part, zeros)
    return _sum_parts(parts[0], parts[1])


# speedup vs baseline: 1.1066x; 1.1066x over previous
import functools
import math

import jax
import jax.numpy as jnp
from jax import lax
from jax.experimental import pallas as pl
from jax.experimental.pallas import tpu as pltpu
from jax.experimental.pallas import tpu_sc as plsc

LANES = 16
C = 128
NC = 2
NS = 16
NW = NC * NS


def _proj_body(x_ref, wq_ref, bq_ref, wk_ref, bk_ref, wv_ref, bv_ref,
               qb_ref, kb_ref, v_ref):
    dn = (((1,), (1,)), ((), ()))
    xb = x_ref[...]
    q = lax.dot_general(xb, wq_ref[...], dn,
                        preferred_element_type=jnp.float32,
                        precision=lax.Precision.HIGHEST) + bq_ref[...]
    k = lax.dot_general(xb, wk_ref[...], dn,
                        preferred_element_type=jnp.float32,
                        precision=lax.Precision.HIGHEST) + bk_ref[...]
    hh = q.shape[1] // 2

    def pack(m):
        bits = lax.shift_right_logical(
            lax.bitcast_convert_type(
                m.astype(jnp.bfloat16).astype(jnp.float32), jnp.int32), 16)
        return bits[:, :hh] | (bits[:, hh:] << 16)

    qb_ref[...] = pack(q)
    kb_ref[...] = pack(k)
    v_ref[...] = lax.dot_general(xb, wv_ref[...], dn,
                                 preferred_element_type=jnp.float32,
                                 precision=lax.Precision.HIGHEST) + bv_ref[...]


def _project(x, W_q, b_q, W_k, b_k, W_v, b_v):
    n, f_in = x.shape
    h = W_q.shape[0]
    blk = 2000
    grid = (n // blk,)
    w_spec = pl.BlockSpec((h, f_in), lambda i: (0, 0))
    b_spec = pl.BlockSpec((1, h), lambda i: (0, 0))
    row_spec = pl.BlockSpec((blk, f_in), lambda i: (i, 0))
    out_spec = pl.BlockSpec((blk, h), lambda i: (i, 0))
    p_spec = pl.BlockSpec((blk, h // 2), lambda i: (i, 0))
    return pl.pallas_call(
        _proj_body,
        grid=grid,
        in_specs=[row_spec, w_spec, b_spec, w_spec, b_spec, w_spec, b_spec],
        out_specs=[p_spec, p_spec, out_spec],
        out_shape=[jax.ShapeDtypeStruct((n, h // 2), jnp.int32),
                   jax.ShapeDtypeStruct((n, h // 2), jnp.int32),
                   jax.ShapeDtypeStruct((n, h), jnp.float32)],
    )(x, W_q, b_q.reshape(1, h), W_k, b_k.reshape(1, h), W_v, b_v.reshape(1, h))


def _make_phase1(n, e, h, g):
    nchunk = e // C
    npair = nchunk // 2
    ppt = -(-npair // NW)
    if ppt % 2:
        ppt += 1
    nsuper = ppt // 2
    base_n = npair // NW
    rem = npair % NW
    mesh = plsc.VectorSubcoreMesh(core_axis_name="c", subcore_axis_name="s")
    inv_sqrt_h = 1.0 / math.sqrt(h)

    @functools.partial(
        pl.kernel,
        out_type=[jax.ShapeDtypeStruct((e,), jnp.float32),
                  jax.ShapeDtypeStruct((NW, g), jnp.float32)],
        mesh=mesh,
        scratch_types=[
            pltpu.VMEM((2 * C,), jnp.int32),
            pltpu.VMEM((2 * C,), jnp.int32),
            pltpu.VMEM((2 * C,), jnp.int32),
            pltpu.VMEM((2 * C,), jnp.int32),
            pltpu.VMEM((C, h // 2), jnp.int32),
            pltpu.VMEM((C, h // 2), jnp.int32),
            pltpu.VMEM((C, h // 2), jnp.int32),
            pltpu.VMEM((C, h // 2), jnp.int32),
            pltpu.VMEM((2 * C,), jnp.float32),
            pltpu.VMEM((2 * C,), jnp.float32),
            pltpu.VMEM((n,), jnp.int32),
            pltpu.VMEM((g * LANES,), jnp.float32),
            pltpu.VMEM((g,), jnp.float32),
            pltpu.VMEM((LANES * LANES,), jnp.float32),
            pltpu.SemaphoreType.DMA,
            pltpu.SemaphoreType.DMA,
            pltpu.SemaphoreType.DMA,
            pltpu.SemaphoreType.DMA,
        ],
        compiler_params=pltpu.CompilerParams(needs_layout_passes=False,
                                             use_tc_tiling_on_sc=False),
    )
    def phase1(q_hbm, k_hbm, src_hbm, dst_hbm, batch_hbm,
               expv_hbm, denom_hbm,
               si0, si1, di0, di1, qv0, qv1, kv0, kv1, ev0, ev1,
               batch_v, dacc_v, dred_v, dots_v,
               sq0, sq1, sk0, sk1):
        si = [si0, si1]
        di = [di0, di1]
        qv = [qv0, qv1]
        kv = [kv0, kv1]
        ev = [ev0, ev1]
        sq = [sq0, sq1]
        sk = [sk0, sk1]
        cid = lax.axis_index("c")
        sid = lax.axis_index("s")
        wid = sid * NC + cid
        iota = jnp.arange(LANES, dtype=jnp.int32)
        zeros16 = jnp.zeros((LANES,), jnp.float32)

        pltpu.sync_copy(batch_hbm, batch_v)

        def zero_body(i, _):
            dacc_v[pl.ds(i * LANES, LANES)] = zeros16
            return _
        lax.fori_loop(0, g, zero_body, None)

        base_pair = wid * base_n + jnp.minimum(wid, rem)
        n_real = jnp.where(wid < rem, base_n + 1, base_n)

        def load_pair_idx(pj, b):
            pc = jnp.minimum(pj, npair - 1)
            pltpu.sync_copy(src_hbm.at[pl.ds(pc * 2 * C, 2 * C)], si[b])
            pltpu.sync_copy(dst_hbm.at[pl.ds(pc * 2 * C, 2 * C)], di[b])

        def issue_gather(b, hh):
            pltpu.async_copy(q_hbm.at[di[b].at[pl.ds(hh * C, C)]],
                             qv[hh], sq[hh])
            pltpu.async_copy(k_hbm.at[si[b].at[pl.ds(hh * C, C)]],
                             kv[hh], sk[hh])

        def wait_gather(b, hh):
            pltpu.make_async_copy(q_hbm.at[di[b].at[pl.ds(hh * C, C)]],
                                  qv[hh], sq[hh]).wait()
            pltpu.make_async_copy(k_hbm.at[si[b].at[pl.ds(hh * C, C)]],
                                  kv[hh], sk[hh]).wait()

        load_pair_idx(base_pair, 0)
        issue_gather(0, 0)

        def super_body(j, _):
            for c in range(4):
                b, hh = c // 2, c % 2
                pair_loc = j * 2 + b
                pair_g = base_pair + pair_loc
                if c == 0:
                    issue_gather(0, 1)
                elif c == 1:
                    load_pair_idx(pair_g + 1, 1)
                    issue_gather(1, 0)
                elif c == 2:
                    issue_gather(1, 1)
                else:
                    @pl.when(j < nsuper - 1)
                    def _prefetch():
                        load_pair_idx(pair_g + 1, 0)
                        issue_gather(0, 0)
                wait_gather(b, hh)
                validf = jnp.where(pair_loc < n_real, 1.0, 0.0)
                vbcast = jnp.full((LANES,), 1.0, jnp.float32) * validf

                def blk_body(blk, _b):
                    for l in range(LANES):
                        ei = blk * LANES + l
                        acc = zeros16
                        for hb in range(h // (2 * LANES)):
                            qi = qv[hh][ei, pl.ds(hb * LANES, LANES)]
                            ki = kv[hh][ei, pl.ds(hb * LANES, LANES)]
                            qlo = plsc.bitcast(qi << 16, jnp.float32)
                            klo = plsc.bitcast(ki << 16, jnp.float32)
                            qhi = plsc.bitcast(qi & jnp.int32(-65536),
                                               jnp.float32)
                            khi = plsc.bitcast(ki & jnp.int32(-65536),
                                               jnp.float32)
                            acc = acc + qlo * klo + qhi * khi
                        dots_v[pl.ds(l * LANES, LANES)] = acc
                    dot = zeros16
                    for jj in range(LANES):
                        dot = dot + plsc.load_gather(dots_v,
                                                     [iota * LANES + jj])
                    sc = jnp.exp(dot * inv_sqrt_h)
                    ev[b][pl.ds(hh * C + blk * LANES, LANES)] = sc
                    src16 = si[b][pl.ds(hh * C + blk * LANES, LANES)]
                    bse16 = plsc.load_gather(batch_v, [src16])
                    plsc.addupdate_scatter(dacc_v, [bse16 * LANES + iota],
                                           sc * vbcast)
                    return _b
                lax.fori_loop(0, C // LANES, blk_body, None)
                if hh == 1:
                    pc = jnp.minimum(pair_g, npair - 1)
                    pltpu.sync_copy(ev[b],
                                    expv_hbm.at[pl.ds(pc * 2 * C, 2 * C)])
            return _
        lax.fori_loop(0, nsuper, super_body, None)

        for blk in range(g // LANES):
            acc = zeros16
            for l in range(LANES):
                acc = acc + plsc.load_gather(
                    dacc_v, [(iota + blk * LANES) * LANES + l])
            dred_v[pl.ds(blk * LANES, LANES)] = acc
        pltpu.sync_copy(dred_v, denom_hbm.at[wid])

    return phase1


def _make_phase2(n, e, h, g):
    nchunk = e // C
    ROWCH = 80
    mesh = plsc.VectorSubcoreMesh(core_axis_name="c", subcore_axis_name="s")

    npair = nchunk // 2
    ppt = -(-npair // NW)
    if ppt % 2:
        ppt += 1
    nsuper = ppt // 2
    base_n = npair // NW
    rem = npair % NW

    @functools.partial(
        pl.kernel,
        out_type=jax.ShapeDtypeStruct((NC, n, h), jnp.float32),
        mesh=mesh,
        scratch_types=[
            pltpu.VMEM((2 * C,), jnp.int32),
            pltpu.VMEM((2 * C,), jnp.int32),
            pltpu.VMEM((2 * C,), jnp.int32),
            pltpu.VMEM((2 * C,), jnp.int32),
            pltpu.VMEM((C,), jnp.int32),
            pltpu.VMEM((C,), jnp.int32),
            pltpu.VMEM((C, h), jnp.float32),
            pltpu.VMEM((C, h), jnp.float32),
            pltpu.VMEM((2 * C,), jnp.float32),
            pltpu.VMEM((2 * C,), jnp.float32),
            pltpu.VMEM((C,), jnp.float32),
            pltpu.VMEM((C,), jnp.float32),
            pltpu.VMEM((n,), jnp.int32),
            pltpu.VMEM((NW, g), jnp.float32),
            pltpu.VMEM((g,), jnp.float32),
            pltpu.VMEM_SHARED((n, h), jnp.float32),
            pltpu.SemaphoreType.DMA,
            pltpu.SemaphoreType.DMA,
            pltpu.SemaphoreType.DMA,
            pltpu.SemaphoreType.DMA,
        ],
        compiler_params=pltpu.CompilerParams(needs_layout_passes=False),
    )
    def phase2(v_hbm, src_hbm, dst_hbm, expv_hbm, batch_hbm, denom_hbm,
               zeros_hbm, part_hbm,
               si0, si1, di0, di1, db0, db1, vv0, vv1, ev0, ev1, av0, av1,
               batch_v, dpart_v, rec_v,
               acc_sh, sv0, sv1, ss0, ss1):
        si = [si0, si1]
        di = [di0, di1]
        db = [db0, db1]
        vv = [vv0, vv1]
        ev = [ev0, ev1]
        av = [av0, av1]
        sv = [sv0, sv1]
        ss = [ss0, ss1]
        cid = lax.axis_index("c")
        sid = lax.axis_index("s")
        wid = sid * NC + cid
        iota = jnp.arange(LANES, dtype=jnp.int32)
        zeros16 = jnp.zeros((LANES,), jnp.float32)

        pltpu.sync_copy(batch_hbm, batch_v)
        pltpu.sync_copy(denom_hbm, dpart_v)
        for blk in range(g // LANES):
            acc = zeros16
            for w in range(NW):
                acc = acc + dpart_v[w, pl.ds(blk * LANES, LANES)]
            rec_v[pl.ds(blk * LANES, LANES)] = 1.0 / (acc + 1e-6)

        nrch = n // ROWCH
        n_z = (nrch - sid + NS - 1) // NS

        def zero_body(i, _):
            r0 = (sid + i * NS) * ROWCH
            pltpu.sync_copy(zeros_hbm.at[pl.ds(r0, ROWCH)],
                            acc_sh.at[pl.ds(r0, ROWCH)])
            return _
        lax.fori_loop(0, n_z, zero_body, None)
        plsc.subcore_barrier()

        base_pair = wid * base_n + jnp.minimum(wid, rem)
        n_real = jnp.where(wid < rem, base_n + 1, base_n)

        def load_pair(pj, b):
            pc = jnp.minimum(pj, npair - 1)
            pltpu.sync_copy(src_hbm.at[pl.ds(pc * 2 * C, 2 * C)], si[b])
            pltpu.sync_copy(dst_hbm.at[pl.ds(pc * 2 * C, 2 * C)], di[b])
            pltpu.sync_copy(expv_hbm.at[pl.ds(pc * 2 * C, 2 * C)], ev[b])

        def issue_vgather(b, hh):
            pltpu.async_copy(v_hbm.at[si[b].at[pl.ds(hh * C, C)]],
                             vv[hh], sv[hh])

        def wait_vgather(b, hh):
            pltpu.make_async_copy(v_hbm.at[si[b].at[pl.ds(hh * C, C)]],
                                  vv[hh], sv[hh]).wait()

        def issue_scatter(hh):
            pltpu.async_copy(vv[hh], acc_sh.at[db[hh]], ss[hh], add=True)

        def wait_scatter(hh):
            pltpu.make_async_copy(vv[hh], acc_sh.at[db[hh]], ss[hh]).wait()

        load_pair(base_pair, 0)
        issue_vgather(0, 0)

        def super_body(j, _):
            for c in range(4):
                b, hh = c // 2, c % 2
                pair_loc = j * 2 + b
                pair_g = base_pair + pair_loc
                if c == 0:
                    @pl.when(j > 0)
                    def _drain():
                        wait_scatter(1)
                    issue_vgather(0, 1)
                elif c == 1:
                    wait_scatter(0)
                    load_pair(pair_g + 1, 1)
                    issue_vgather(1, 0)
                elif c == 2:
                    wait_scatter(1)
                    issue_vgather(1, 1)
                else:
                    @pl.when(j < nsuper - 1)
                    def _prefetch():
                        wait_scatter(0)
                        load_pair(pair_g + 1, 0)
                        issue_vgather(0, 0)
                wait_vgather(b, hh)
                validf = jnp.where(pair_loc < n_real, 1.0, 0.0)
                vbcast = jnp.full((LANES,), 1.0, jnp.float32) * validf
                for blk in range(C // LANES):
                    off = hh * C + blk * LANES
                    src16 = si[b][pl.ds(off, LANES)]
                    bse16 = plsc.load_gather(batch_v, [src16])
                    rd16 = plsc.load_gather(rec_v, [bse16])
                    av[hh][pl.ds(blk * LANES, LANES)] = \
                        ev[b][pl.ds(off, LANES)] * rd16 * vbcast
                    db[hh][pl.ds(blk * LANES, LANES)] = di[b][pl.ds(off, LANES)]

                def scale_body(blk, _s):
                    av16 = av[hh][pl.ds(blk * LANES, LANES)]
                    for l in range(LANES):
                        ei = blk * LANES + l
                        a_s = av16[l]
                        for hb in range(h // LANES):
                            vv[hh][ei, pl.ds(hb * LANES, LANES)] = \
                                vv[hh][ei, pl.ds(hb * LANES, LANES)] * a_s
                    return _s
                lax.fori_loop(0, C // LANES, scale_body, None)
                issue_scatter(hh)
            return _
        lax.fori_loop(0, nsuper, super_body, None)

        wait_scatter(0)
        wait_scatter(1)
        plsc.subcore_barrier()

        def out_body(i, _):
            r0 = (sid + i * NS) * ROWCH
            pltpu.sync_copy(acc_sh.at[pl.ds(r0, ROWCH)],
                            part_hbm.at[cid, pl.ds(r0, ROWCH)])
            return _
        lax.fori_loop(0, n_z, out_body, None)

    return phase2


def _sum_body(a_ref, b_ref, o_ref):
    o_ref[...] = a_ref[...] + b_ref[...]


def _sum_parts(p0, p1):
    n, h = p0.shape
    blk = 2000
    spec = pl.BlockSpec((blk, h), lambda i: (i, 0))
    return pl.pallas_call(
        _sum_body,
        grid=(n // blk,),
        in_specs=[spec, spec],
        out_specs=spec,
        out_shape=jax.ShapeDtypeStruct((n, h), jnp.float32),
    )(p0, p1)


def kernel(x, edge_index, batch, W_q, b_q, W_k, b_k, W_v, b_v):
    n, f_in = x.shape
    e = edge_index.shape[1]
    h = W_q.shape[0]
    g = 64

    q32, k32, v = _project(x, W_q, b_q, W_k, b_k, W_v, b_v)
    src = edge_index[0]
    dst = edge_index[1]

    expv, denom_part = _make_phase1(n, e, h, g)(q32, k32, src, dst, batch)
    zeros = jnp.zeros((n, h), jnp.float32)
    parts = _make_phase2(n, e, h, g)(v, src, dst, expv, batch, denom_part, zeros)
    return _sum_parts(parts[0], parts[1])


# speedup vs baseline: 1.1497x; 1.0390x over previous
import functools
import math

import jax
import jax.numpy as jnp
from jax import lax
from jax.experimental import pallas as pl
from jax.experimental.pallas import tpu as pltpu
from jax.experimental.pallas import tpu_sc as plsc

LANES = 16
C = 128
CE = 256
NC = 2
NS = 16
NW = NC * NS


def _proj_body(x_ref, wq_ref, bq_ref, wk_ref, bk_ref, wv_ref, bv_ref,
               qb_ref, kb_ref, v_ref):
    dn = (((1,), (1,)), ((), ()))
    xb = x_ref[...]
    q = lax.dot_general(xb, wq_ref[...], dn,
                        preferred_element_type=jnp.float32,
                        precision=lax.Precision.HIGHEST) + bq_ref[...]
    k = lax.dot_general(xb, wk_ref[...], dn,
                        preferred_element_type=jnp.float32,
                        precision=lax.Precision.HIGHEST) + bk_ref[...]
    hh = q.shape[1] // 2

    def pack(m):
        bits = lax.shift_right_logical(
            lax.bitcast_convert_type(
                m.astype(jnp.bfloat16).astype(jnp.float32), jnp.int32), 16)
        return bits[:, :hh] | (bits[:, hh:] << 16)

    qb_ref[...] = pack(q)
    kb_ref[...] = pack(k)
    v_ref[...] = lax.dot_general(xb, wv_ref[...], dn,
                                 preferred_element_type=jnp.float32,
                                 precision=lax.Precision.HIGHEST) + bv_ref[...]


def _project(x, W_q, b_q, W_k, b_k, W_v, b_v):
    n, f_in = x.shape
    h = W_q.shape[0]
    blk = 2000
    grid = (n // blk,)
    w_spec = pl.BlockSpec((h, f_in), lambda i: (0, 0))
    b_spec = pl.BlockSpec((1, h), lambda i: (0, 0))
    row_spec = pl.BlockSpec((blk, f_in), lambda i: (i, 0))
    out_spec = pl.BlockSpec((blk, h), lambda i: (i, 0))
    p_spec = pl.BlockSpec((blk, h // 2), lambda i: (i, 0))
    return pl.pallas_call(
        _proj_body,
        grid=grid,
        in_specs=[row_spec, w_spec, b_spec, w_spec, b_spec, w_spec, b_spec],
        out_specs=[p_spec, p_spec, out_spec],
        out_shape=[jax.ShapeDtypeStruct((n, h // 2), jnp.int32),
                   jax.ShapeDtypeStruct((n, h // 2), jnp.int32),
                   jax.ShapeDtypeStruct((n, h), jnp.float32)],
    )(x, W_q, b_q.reshape(1, h), W_k, b_k.reshape(1, h), W_v, b_v.reshape(1, h))


def _make_phase1(n, e, h, g):
    nchunk = e // CE
    npair = nchunk // 2
    ppt = -(-npair // NW)
    if ppt % 2:
        ppt += 1
    nsuper = ppt // 2
    base_n = npair // NW
    rem = npair % NW
    mesh = plsc.VectorSubcoreMesh(core_axis_name="c", subcore_axis_name="s")
    inv_sqrt_h = 1.0 / math.sqrt(h)

    @functools.partial(
        pl.kernel,
        out_type=[jax.ShapeDtypeStruct((e,), jnp.float32),
                  jax.ShapeDtypeStruct((NW, g), jnp.float32)],
        mesh=mesh,
        scratch_types=[
            pltpu.VMEM((2 * CE,), jnp.int32),
            pltpu.VMEM((2 * CE,), jnp.int32),
            pltpu.VMEM((2 * CE,), jnp.int32),
            pltpu.VMEM((2 * CE,), jnp.int32),
            pltpu.VMEM((CE, h // 2), jnp.int32),
            pltpu.VMEM((CE, h // 2), jnp.int32),
            pltpu.VMEM((CE, h // 2), jnp.int32),
            pltpu.VMEM((CE, h // 2), jnp.int32),
            pltpu.VMEM((2 * CE,), jnp.float32),
            pltpu.VMEM((2 * CE,), jnp.float32),
            pltpu.VMEM((n,), jnp.int32),
            pltpu.VMEM((g * LANES,), jnp.float32),
            pltpu.VMEM((g,), jnp.float32),
            pltpu.VMEM((LANES * LANES,), jnp.float32),
            pltpu.SemaphoreType.DMA,
            pltpu.SemaphoreType.DMA,
            pltpu.SemaphoreType.DMA,
            pltpu.SemaphoreType.DMA,
        ],
        compiler_params=pltpu.CompilerParams(needs_layout_passes=False,
                                             use_tc_tiling_on_sc=False),
    )
    def phase1(q_hbm, k_hbm, src_hbm, dst_hbm, batch_hbm,
               expv_hbm, denom_hbm,
               si0, si1, di0, di1, qv0, qv1, kv0, kv1, ev0, ev1,
               batch_v, dacc_v, dred_v, dots_v,
               sq0, sq1, sk0, sk1):
        si = [si0, si1]
        di = [di0, di1]
        qv = [qv0, qv1]
        kv = [kv0, kv1]
        ev = [ev0, ev1]
        sq = [sq0, sq1]
        sk = [sk0, sk1]
        cid = lax.axis_index("c")
        sid = lax.axis_index("s")
        wid = sid * NC + cid
        iota = jnp.arange(LANES, dtype=jnp.int32)
        zeros16 = jnp.zeros((LANES,), jnp.float32)

        pltpu.sync_copy(batch_hbm, batch_v)

        def zero_body(i, _):
            dacc_v[pl.ds(i * LANES, LANES)] = zeros16
            return _
        lax.fori_loop(0, g, zero_body, None)

        base_pair = wid * base_n + jnp.minimum(wid, rem)
        n_real = jnp.where(wid < rem, base_n + 1, base_n)

        def load_pair_idx(pj, b):
            pc = jnp.minimum(pj, npair - 1)
            pltpu.sync_copy(src_hbm.at[pl.ds(pc * 2 * CE, 2 * CE)], si[b])
            pltpu.sync_copy(dst_hbm.at[pl.ds(pc * 2 * CE, 2 * CE)], di[b])

        def issue_gather(b, hh):
            for hf in range(CE // C):
                pltpu.async_copy(
                    q_hbm.at[di[b].at[pl.ds(hh * CE + hf * C, C)]],
                    qv[hh].at[pl.ds(hf * C, C)], sq[hh])
                pltpu.async_copy(
                    k_hbm.at[si[b].at[pl.ds(hh * CE + hf * C, C)]],
                    kv[hh].at[pl.ds(hf * C, C)], sk[hh])

        def wait_gather(b, hh):
            for hf in range(CE // C):
                pltpu.make_async_copy(
                    q_hbm.at[di[b].at[pl.ds(hh * CE + hf * C, C)]],
                    qv[hh].at[pl.ds(hf * C, C)], sq[hh]).wait()
                pltpu.make_async_copy(
                    k_hbm.at[si[b].at[pl.ds(hh * CE + hf * C, C)]],
                    kv[hh].at[pl.ds(hf * C, C)], sk[hh]).wait()

        load_pair_idx(base_pair, 0)
        issue_gather(0, 0)

        def super_body(j, _):
            for c in range(4):
                b, hh = c // 2, c % 2
                pair_loc = j * 2 + b
                pair_g = base_pair + pair_loc
                if c == 0:
                    issue_gather(0, 1)
                elif c == 1:
                    load_pair_idx(pair_g + 1, 1)
                    issue_gather(1, 0)
                elif c == 2:
                    issue_gather(1, 1)
                else:
                    @pl.when(j < nsuper - 1)
                    def _prefetch():
                        load_pair_idx(pair_g + 1, 0)
                        issue_gather(0, 0)
                wait_gather(b, hh)
                validf = jnp.where(pair_loc < n_real, 1.0, 0.0)
                vbcast = jnp.full((LANES,), 1.0, jnp.float32) * validf

                def blk_body(blk, _b):
                    for l in range(LANES):
                        ei = blk * LANES + l
                        acc = zeros16
                        for hb in range(h // (2 * LANES)):
                            qi = qv[hh][ei, pl.ds(hb * LANES, LANES)]
                            ki = kv[hh][ei, pl.ds(hb * LANES, LANES)]
                            qlo = plsc.bitcast(qi << 16, jnp.float32)
                            klo = plsc.bitcast(ki << 16, jnp.float32)
                            qhi = plsc.bitcast(qi & jnp.int32(-65536),
                                               jnp.float32)
                            khi = plsc.bitcast(ki & jnp.int32(-65536),
                                               jnp.float32)
                            acc = acc + qlo * klo + qhi * khi
                        dots_v[pl.ds(l * LANES, LANES)] = acc
                    dot = zeros16
                    for jj in range(LANES):
                        dot = dot + plsc.load_gather(dots_v,
                                                     [iota * LANES + jj])
                    sc = jnp.exp(dot * inv_sqrt_h)
                    ev[b][pl.ds(hh * CE + blk * LANES, LANES)] = sc
                    src16 = si[b][pl.ds(hh * CE + blk * LANES, LANES)]
                    bse16 = plsc.load_gather(batch_v, [src16])
                    plsc.addupdate_scatter(dacc_v, [bse16 * LANES + iota],
                                           sc * vbcast)
                    return _b
                lax.fori_loop(0, CE // LANES, blk_body, None)
                if hh == 1:
                    pc = jnp.minimum(pair_g, npair - 1)
                    pltpu.sync_copy(ev[b],
                                    expv_hbm.at[pl.ds(pc * 2 * CE, 2 * CE)])
            return _
        lax.fori_loop(0, nsuper, super_body, None)

        for blk in range(g // LANES):
            acc = zeros16
            for l in range(LANES):
                acc = acc + plsc.load_gather(
                    dacc_v, [(iota + blk * LANES) * LANES + l])
            dred_v[pl.ds(blk * LANES, LANES)] = acc
        pltpu.sync_copy(dred_v, denom_hbm.at[wid])

    return phase1


def _make_phase2(n, e, h, g):
    CE = C
    nchunk = e // CE
    ROWCH = 80
    mesh = plsc.VectorSubcoreMesh(core_axis_name="c", subcore_axis_name="s")

    npair = nchunk // 2
    ppt = -(-npair // NW)
    if ppt % 2:
        ppt += 1
    nsuper = ppt // 2
    base_n = npair // NW
    rem = npair % NW

    @functools.partial(
        pl.kernel,
        out_type=jax.ShapeDtypeStruct((NC, n, h), jnp.float32),
        mesh=mesh,
        scratch_types=[
            pltpu.VMEM((2 * CE,), jnp.int32),
            pltpu.VMEM((2 * CE,), jnp.int32),
            pltpu.VMEM((2 * CE,), jnp.int32),
            pltpu.VMEM((2 * CE,), jnp.int32),
            pltpu.VMEM((C,), jnp.int32),
            pltpu.VMEM((C,), jnp.int32),
            pltpu.VMEM((C,), jnp.int32),
            pltpu.VMEM((C,), jnp.int32),
            pltpu.VMEM((CE, h), jnp.float32),
            pltpu.VMEM((CE, h), jnp.float32),
            pltpu.VMEM((2 * CE,), jnp.float32),
            pltpu.VMEM((2 * CE,), jnp.float32),
            pltpu.VMEM((CE,), jnp.float32),
            pltpu.VMEM((CE,), jnp.float32),
            pltpu.VMEM((n,), jnp.int32),
            pltpu.VMEM((NW, g), jnp.float32),
            pltpu.VMEM((g,), jnp.float32),
            pltpu.VMEM_SHARED((n, h), jnp.float32),
            pltpu.SemaphoreType.DMA,
            pltpu.SemaphoreType.DMA,
            pltpu.SemaphoreType.DMA,
            pltpu.SemaphoreType.DMA,
        ],
        compiler_params=pltpu.CompilerParams(needs_layout_passes=False),
    )
    def phase2(v_hbm, src_hbm, dst_hbm, expv_hbm, batch_hbm, denom_hbm,
               zeros_hbm, part_hbm,
               si0, si1, di0, di1, db00, db01, db10, db11,
               vv0, vv1, ev0, ev1, av0, av1,
               batch_v, dpart_v, rec_v,
               acc_sh, sv0, sv1, ss0, ss1):
        si = [si0, si1]
        di = [di0, di1]
        db = [[db00, db01], [db10, db11]]
        vv = [vv0, vv1]
        ev = [ev0, ev1]
        av = [av0, av1]
        sv = [sv0, sv1]
        ss = [ss0, ss1]
        cid = lax.axis_index("c")
        sid = lax.axis_index("s")
        wid = sid * NC + cid
        iota = jnp.arange(LANES, dtype=jnp.int32)
        zeros16 = jnp.zeros((LANES,), jnp.float32)

        pltpu.sync_copy(batch_hbm, batch_v)
        pltpu.sync_copy(denom_hbm, dpart_v)
        for blk in range(g // LANES):
            acc = zeros16
            for w in range(NW):
                acc = acc + dpart_v[w, pl.ds(blk * LANES, LANES)]
            rec_v[pl.ds(blk * LANES, LANES)] = 1.0 / (acc + 1e-6)

        nrch = n // ROWCH
        n_z = (nrch - sid + NS - 1) // NS

        def zero_body(i, _):
            r0 = (sid + i * NS) * ROWCH
            pltpu.sync_copy(zeros_hbm.at[pl.ds(r0, ROWCH)],
                            acc_sh.at[pl.ds(r0, ROWCH)])
            return _
        lax.fori_loop(0, n_z, zero_body, None)
        plsc.subcore_barrier()

        base_pair = wid * base_n + jnp.minimum(wid, rem)
        n_real = jnp.where(wid < rem, base_n + 1, base_n)

        def load_pair(pj, b):
            pc = jnp.minimum(pj, npair - 1)
            pltpu.sync_copy(src_hbm.at[pl.ds(pc * 2 * CE, 2 * CE)], si[b])
            pltpu.sync_copy(dst_hbm.at[pl.ds(pc * 2 * CE, 2 * CE)], di[b])
            pltpu.sync_copy(expv_hbm.at[pl.ds(pc * 2 * CE, 2 * CE)], ev[b])

        def issue_vgather(b, hh):
            for hf in range(CE // C):
                pltpu.async_copy(
                    v_hbm.at[si[b].at[pl.ds(hh * CE + hf * C, C)]],
                    vv[hh].at[pl.ds(hf * C, C)], sv[hh])

        def wait_vgather(b, hh):
            for hf in range(CE // C):
                pltpu.make_async_copy(
                    v_hbm.at[si[b].at[pl.ds(hh * CE + hf * C, C)]],
                    vv[hh].at[pl.ds(hf * C, C)], sv[hh]).wait()

        def issue_scatter(hh):
            for hf in range(CE // C):
                pltpu.async_copy(vv[hh].at[pl.ds(hf * C, C)],
                                 acc_sh.at[db[hh][hf]], ss[hh], add=True)

        def wait_scatter(hh):
            for hf in range(CE // C):
                pltpu.make_async_copy(vv[hh].at[pl.ds(hf * C, C)],
                                      acc_sh.at[db[hh][hf]], ss[hh]).wait()

        load_pair(base_pair, 0)
        issue_vgather(0, 0)

        def super_body(j, _):
            for c in range(4):
                b, hh = c // 2, c % 2
                pair_loc = j * 2 + b
                pair_g = base_pair + pair_loc
                if c == 0:
                    @pl.when(j > 0)
                    def _drain():
                        wait_scatter(1)
                    issue_vgather(0, 1)
                elif c == 1:
                    wait_scatter(0)
                    load_pair(pair_g + 1, 1)
                    issue_vgather(1, 0)
                elif c == 2:
                    wait_scatter(1)
                    issue_vgather(1, 1)
                else:
                    @pl.when(j < nsuper - 1)
                    def _prefetch():
                        wait_scatter(0)
                        load_pair(pair_g + 1, 0)
                        issue_vgather(0, 0)
                wait_vgather(b, hh)
                validf = jnp.where(pair_loc < n_real, 1.0, 0.0)
                vbcast = jnp.full((LANES,), 1.0, jnp.float32) * validf
                for blk in range(CE // LANES):
                    off = hh * CE + blk * LANES
                    src16 = si[b][pl.ds(off, LANES)]
                    bse16 = plsc.load_gather(batch_v, [src16])
                    rd16 = plsc.load_gather(rec_v, [bse16])
                    av[hh][pl.ds(blk * LANES, LANES)] = \
                        ev[b][pl.ds(off, LANES)] * rd16 * vbcast
                    db[hh][blk // (C // LANES)][
                        pl.ds((blk % (C // LANES)) * LANES, LANES)] = \
                        di[b][pl.ds(off, LANES)]

                def scale_body(blk, _s):
                    av16 = av[hh][pl.ds(blk * LANES, LANES)]
                    for l in range(LANES):
                        ei = blk * LANES + l
                        a_s = av16[l]
                        for hb in range(h // LANES):
                            vv[hh][ei, pl.ds(hb * LANES, LANES)] = \
                                vv[hh][ei, pl.ds(hb * LANES, LANES)] * a_s
                    return _s
                lax.fori_loop(0, CE // LANES, scale_body, None)
                issue_scatter(hh)
            return _
        lax.fori_loop(0, nsuper, super_body, None)

        wait_scatter(0)
        wait_scatter(1)
        plsc.subcore_barrier()

        def out_body(i, _):
            r0 = (sid + i * NS) * ROWCH
            pltpu.sync_copy(acc_sh.at[pl.ds(r0, ROWCH)],
                            part_hbm.at[cid, pl.ds(r0, ROWCH)])
            return _
        lax.fori_loop(0, n_z, out_body, None)

    return phase2


def _sum_body(a_ref, b_ref, o_ref):
    o_ref[...] = a_ref[...] + b_ref[...]


def _sum_parts(p0, p1):
    n, h = p0.shape
    blk = 2000
    spec = pl.BlockSpec((blk, h), lambda i: (i, 0))
    return pl.pallas_call(
        _sum_body,
        grid=(n // blk,),
        in_specs=[spec, spec],
        out_specs=spec,
        out_shape=jax.ShapeDtypeStruct((n, h), jnp.float32),
    )(p0, p1)


def kernel(x, edge_index, batch, W_q, b_q, W_k, b_k, W_v, b_v):
    n, f_in = x.shape
    e = edge_index.shape[1]
    h = W_q.shape[0]
    g = 64

    q32, k32, v = _project(x, W_q, b_q, W_k, b_k, W_v, b_v)
    src = edge_index[0]
    dst = edge_index[1]

    expv, denom_part = _make_phase1(n, e, h, g)(q32, k32, src, dst, batch)
    zeros = jnp.zeros((n, h), jnp.float32)
    parts = _make_phase2(n, e, h, g)(v, src, dst, expv, batch, denom_part, zeros)
    return _sum_parts(parts[0], parts[1])
